# baseline (speedup 1.0000x reference)
import functools

import jax
import jax.numpy as jnp
from jax import lax
from jax.experimental import pallas as pl
from jax.experimental.pallas import tpu as pltpu
from jax.experimental.pallas import tpu_sc as plsc

B = 2048
D = 2048
H = 2048
O = 2048
E = 8
ES = 2
K = 2

GB = 256
BM = 128
RP = K * B + E * BM
RB = RP // BM
TB = B // GB

NWORK = 32
NCH = 4


def _gate_body(x_ref, gw_ref, gb_ref, sc_ref, tw_ref, tid_ref):
    xb = x_ref[...]
    logits = jnp.dot(xb, gw_ref[...], preferred_element_type=jnp.float32)
    logits = logits + gb_ref[...]
    c = lax.broadcasted_iota(jnp.int32, (GB, 128), 1)
    lm = jnp.where(c < E + ES, logits, -1e30)
    m = jnp.max(lm, axis=1, keepdims=True)
    p = jnp.exp(lm - m)
    scores = p / jnp.sum(p, axis=1, keepdims=True)
    sc_ref[...] = scores

    es = jnp.where((c >= ES) & (c < E + ES), scores, -1.0)
    m1 = jnp.max(es, axis=1, keepdims=True)
    i1 = jnp.min(jnp.where(es == m1, c, 999), axis=1, keepdims=True)
    es2 = jnp.where(c == i1, -1.0, es)
    m2 = jnp.max(es2, axis=1, keepdims=True)
    i2 = jnp.min(jnp.where(es2 == m2, c, 999), axis=1, keepdims=True)
    tw_ref[...] = jnp.where(c == 0, m1, 0.0) + jnp.where(c == 1, m2, 0.0)
    tid_ref[...] = (jnp.where(c == 0, i1 - ES, 0)
                    + jnp.where(c == 1, i2 - ES, 0))


def _gate(x, gate_W, gate_b):
    gw = jnp.zeros((D, 128), jnp.float32).at[:, :E + ES].set(gate_W)
    gb = jnp.zeros((1, 128), jnp.float32).at[0, :E + ES].set(gate_b)
    return pl.pallas_call(
        _gate_body,
        grid=(TB,),
        in_specs=[
            pl.BlockSpec((GB, D), lambda i: (i, 0)),
            pl.BlockSpec((D, 128), lambda i: (0, 0)),
            pl.BlockSpec((1, 128), lambda i: (0, 0)),
        ],
        out_specs=[
            pl.BlockSpec((GB, 128), lambda i: (i, 0)),
            pl.BlockSpec((GB, 128), lambda i: (i, 0)),
            pl.BlockSpec((GB, 128), lambda i: (i, 0)),
        ],
        out_shape=[
            jax.ShapeDtypeStruct((B, 128), jnp.float32),
            jax.ShapeDtypeStruct((B, 128), jnp.float32),
            jax.ShapeDtypeStruct((B, 128), jnp.int32),
        ],
    )(x, gw, gb)


def _route_meta(tid2):
    e_flat = tid2.reshape(-1)
    oh = (e_flat[:, None] == jnp.arange(E)[None, :]).astype(jnp.int32)
    counts = jnp.sum(oh, axis=0)
    rank_all = jnp.cumsum(oh, axis=0) - oh
    rank = jnp.take_along_axis(rank_all, e_flat[:, None], axis=1)[:, 0]
    pc = ((counts + BM - 1) // BM) * BM
    cum_pc = jnp.cumsum(pc)
    off = cum_pc - pc
    pos = off[e_flat] + rank
    nblk = cum_pc[-1] // BM
    mb = jnp.arange(RB, dtype=jnp.int32)
    valid = (mb < nblk).astype(jnp.int32)
    mmap = jnp.minimum(mb, nblk - 1)
    gmap = jnp.minimum(
        jnp.searchsorted(cum_pc, mmap * BM, side='right').astype(jnp.int32),
        E - 1)
    pos2 = pos.reshape(B, K)
    pos_cat = jnp.concatenate([pos2[:, 0], pos2[:, 1]]).astype(jnp.int32)
    return pos_cat, gmap, mmap, valid


def _sc_dispatch(x, pos_cat):
    tpw = B // NWORK
    ch = 16
    nc = tpw // ch
    mesh = plsc.VectorSubcoreMesh(core_axis_name="c", subcore_axis_name="s")

    @functools.partial(
        pl.kernel,
        out_type=jax.ShapeDtypeStruct((RP, D), jnp.float32),
        mesh=mesh,
        scratch_types=[
            pltpu.VMEM((2, ch, D), jnp.float32),
            pltpu.VMEM((2, ch), jnp.int32),
            pltpu.VMEM((2, ch), jnp.int32),
            pltpu.SemaphoreType.DMA,
            pltpu.SemaphoreType.DMA,
        ],
    )
    def sk(x_hbm, pos_hbm, out_hbm, rows_v, i0_v, i1_v, sem0, sem1):
        wid = lax.axis_index("s") * 2 + lax.axis_index("c")
        base = wid * tpw
        sems = (sem0, sem1)
        pending = [[], []]
        for c in range(nc):
            b = c % 2
            for hnd in pending[b]:
                hnd.wait()
            pending[b] = []
            tb = base + c * ch
            pltpu.sync_copy(x_hbm.at[pl.ds(tb, ch)], rows_v.at[b])
            pltpu.sync_copy(pos_hbm.at[pl.ds(tb, ch)], i0_v.at[b])
            pltpu.sync_copy(pos_hbm.at[pl.ds(B + tb, ch)], i1_v.at[b])
            pending[b].append(pltpu.async_copy(
                rows_v.at[b], out_hbm.at[i0_v.at[b]], sems[b]))
            pending[b].append(pltpu.async_copy(
                rows_v.at[b], out_hbm.at[i1_v.at[b]], sems[b]))
        for plist in pending:
            for hnd in plist:
                hnd.wait()

    return sk(x, pos_cat)


def _sc_gather(table, idx):
    n_idx = idx.shape[0]
    npw = n_idx // NWORK
    ch = npw // NCH
    mesh = plsc.VectorSubcoreMesh(core_axis_name="c", subcore_axis_name="s")

    @functools.partial(
        pl.kernel,
        out_type=jax.ShapeDtypeStruct((n_idx, D), jnp.float32),
        mesh=mesh,
        scratch_types=[
            pltpu.VMEM((npw,), jnp.int32),
            pltpu.VMEM((ch, D), jnp.float32),
            pltpu.SemaphoreType.DMA,
        ],
    )
    def gk(table_hbm, idx_hbm, out_hbm, idx_v, rows_v, sem):
        wid = lax.axis_index("s") * 2 + lax.axis_index("c")
        base = wid * npw
        pltpu.sync_copy(idx_hbm.at[pl.ds(base, npw)], idx_v)
        for ci in range(NCH):
            pltpu.async_copy(
                table_hbm.at[idx_v.at[pl.ds(ci * ch, ch)]],
                rows_v, sem).wait()
            pltpu.sync_copy(rows_v, out_hbm.at[pl.ds(base + ci * ch, ch)])

    return gk(table, idx)


def _glayer_body(gmap_ref, mmap_ref, valid_ref, x_ref, w_ref, b_ref, out_ref,
                 *, relu):
    mb = pl.program_id(0)
    v = valid_ref[mb] == 1

    @pl.when(v)
    def _():
        y = jnp.dot(x_ref[...], w_ref[0], preferred_element_type=jnp.float32)
        y = y + b_ref[0]
        if relu:
            y = jnp.maximum(y, 0.0)
        out_ref[...] = y


def _glayer(xin, W, b, gmap, mmap, valid, relu):
    n_out = W.shape[2]
    grid_spec = pltpu.PrefetchScalarGridSpec(
        num_scalar_prefetch=3,
        grid=(RB,),
        in_specs=[
            pl.BlockSpec((BM, W.shape[1]), lambda mb, g, m, v: (m[mb], 0)),
            pl.BlockSpec((1, W.shape[1], n_out),
                         lambda mb, g, m, v: (g[mb], 0, 0)),
            pl.BlockSpec((1, 1, n_out), lambda mb, g, m, v: (g[mb], 0, 0)),
        ],
        out_specs=pl.BlockSpec((BM, n_out), lambda mb, g, m, v: (m[mb], 0)),
    )
    return pl.pallas_call(
        functools.partial(_glayer_body, relu=relu),
        grid_spec=grid_spec,
        out_shape=jax.ShapeDtypeStruct((RP, n_out), jnp.float32),
    )(gmap, mmap, valid, xin, W, b.reshape(E, 1, n_out))


def _slayer(xin, W, b, relu, x_3d):
    n_in, n_out = W.shape[1], W.shape[2]
    x_spec = (pl.BlockSpec((1, GB, n_in), lambda s, tb: (s, tb, 0)) if x_3d
              else pl.BlockSpec((GB, n_in), lambda s, tb: (tb, 0)))

    def body(x_ref, w_ref, b_ref, out_ref):
        xb = x_ref[0] if x_3d else x_ref[...]
        y = jnp.dot(xb, w_ref[0], preferred_element_type=jnp.float32)
        y = y + b_ref[0]
        if relu:
            y = jnp.maximum(y, 0.0)
        out_ref[0] = y

    return pl.pallas_call(
        body,
        grid=(ES, TB),
        in_specs=[
            x_spec,
            pl.BlockSpec((1, n_in, n_out), lambda s, tb: (s, 0, 0)),
            pl.BlockSpec((1, 1, n_out), lambda s, tb: (s, 0, 0)),
        ],
        out_specs=pl.BlockSpec((1, GB, n_out), lambda s, tb: (s, tb, 0)),
        out_shape=jax.ShapeDtypeStruct((ES, B, n_out), jnp.float32),
    )(xin, W, b.reshape(ES, 1, n_out))


def _final_body(sc_ref, tw_ref, y0_ref, y1_ref, g0_ref, g1_ref, out_ref):
    sw0 = sc_ref[...][:, 0:1]
    sw1 = sc_ref[...][:, 1:2]
    w0 = tw_ref[...][:, 0:1]
    w1 = tw_ref[...][:, 1:2]
    out_ref[...] = (sw0 * y0_ref[0] + sw1 * y1_ref[0]
                    + w0 * g0_ref[...] + w1 * g1_ref[...])


def _final(scores, tw, ys, g):
    return pl.pallas_call(
        _final_body,
        grid=(TB,),
        in_specs=[
            pl.BlockSpec((GB, 128), lambda tb: (tb, 0)),
            pl.BlockSpec((GB, 128), lambda tb: (tb, 0)),
            pl.BlockSpec((1, GB, O), lambda tb: (0, tb, 0)),
            pl.BlockSpec((1, GB, O), lambda tb: (1, tb, 0)),
            pl.BlockSpec((GB, O), lambda tb: (tb, 0)),
            pl.BlockSpec((GB, O), lambda tb: (tb + TB, 0)),
        ],
        out_specs=pl.BlockSpec((GB, O), lambda tb: (tb, 0)),
        out_shape=jax.ShapeDtypeStruct((B, O), jnp.float32),
    )(scores, tw, ys, ys, g, g)


def kernel(x, gate_W, gate_b, W1, b1, W2, b2, Ws1, bs1, Ws2, bs2):
    scores, tw, tid = _gate(x, gate_W, gate_b)
    pos_cat, gmap, mmap, valid = _route_meta(tid[:, :K])
    xr = _sc_dispatch(x, pos_cat)
    h1r = _glayer(xr, W1, b1, gmap, mmap, valid, relu=True)
    y2r = _glayer(h1r, W2, b2, gmap, mmap, valid, relu=False)
    g = _sc_gather(y2r, pos_cat)
    hs = _slayer(x, Ws1, bs1, relu=True, x_3d=False)
    ys = _slayer(hs, Ws2, bs2, relu=False, x_3d=True)
    return _final(scores, tw, ys, g)


# speedup vs baseline: 1.0450x; 1.0450x over previous
import functools

import jax
import jax.numpy as jnp
from jax import lax
from jax.experimental import pallas as pl
from jax.experimental.pallas import tpu as pltpu
from jax.experimental.pallas import tpu_sc as plsc

B = 2048
D = 2048
H = 2048
O = 2048
E = 8
ES = 2
K = 2

GB = 256
BM = 256
RP = K * B + E * BM
RB = RP // BM
TB = B // GB

NWORK = 32
NCH = 8


def _gate_body(x_ref, gw_ref, gb_ref, sc_ref, tw_ref, tid_ref):
    xb = x_ref[...]
    logits = jnp.dot(xb, gw_ref[...], preferred_element_type=jnp.float32)
    logits = logits + gb_ref[...]
    c = lax.broadcasted_iota(jnp.int32, (GB, 128), 1)
    lm = jnp.where(c < E + ES, logits, -1e30)
    m = jnp.max(lm, axis=1, keepdims=True)
    p = jnp.exp(lm - m)
    scores = p / jnp.sum(p, axis=1, keepdims=True)
    sc_ref[...] = scores

    es = jnp.where((c >= ES) & (c < E + ES), scores, -1.0)
    m1 = jnp.max(es, axis=1, keepdims=True)
    i1 = jnp.min(jnp.where(es == m1, c, 999), axis=1, keepdims=True)
    es2 = jnp.where(c == i1, -1.0, es)
    m2 = jnp.max(es2, axis=1, keepdims=True)
    i2 = jnp.min(jnp.where(es2 == m2, c, 999), axis=1, keepdims=True)
    tw_ref[...] = jnp.where(c == 0, m1, 0.0) + jnp.where(c == 1, m2, 0.0)
    tid_ref[...] = (jnp.where(c == 0, i1 - ES, 0)
                    + jnp.where(c == 1, i2 - ES, 0))


def _gate(x, gate_W, gate_b):
    gw = jnp.zeros((D, 128), jnp.float32).at[:, :E + ES].set(gate_W)
    gb = jnp.zeros((1, 128), jnp.float32).at[0, :E + ES].set(gate_b)
    return pl.pallas_call(
        _gate_body,
        grid=(TB,),
        in_specs=[
            pl.BlockSpec((GB, D), lambda i: (i, 0)),
            pl.BlockSpec((D, 128), lambda i: (0, 0)),
            pl.BlockSpec((1, 128), lambda i: (0, 0)),
        ],
        out_specs=[
            pl.BlockSpec((GB, 128), lambda i: (i, 0)),
            pl.BlockSpec((GB, 128), lambda i: (i, 0)),
            pl.BlockSpec((GB, 128), lambda i: (i, 0)),
        ],
        out_shape=[
            jax.ShapeDtypeStruct((B, 128), jnp.float32),
            jax.ShapeDtypeStruct((B, 128), jnp.float32),
            jax.ShapeDtypeStruct((B, 128), jnp.int32),
        ],
    )(x, gw, gb)


def _route_meta(tid2):
    e_flat = tid2.reshape(-1)
    oh = (e_flat[:, None] == jnp.arange(E)[None, :]).astype(jnp.int32)
    counts = jnp.sum(oh, axis=0)
    rank_all = jnp.cumsum(oh, axis=0) - oh
    rank = jnp.take_along_axis(rank_all, e_flat[:, None], axis=1)[:, 0]
    pc = ((counts + BM - 1) // BM) * BM
    cum_pc = jnp.cumsum(pc)
    off = cum_pc - pc
    pos = off[e_flat] + rank
    nblk = cum_pc[-1] // BM
    mb = jnp.arange(RB, dtype=jnp.int32)
    valid = (mb < nblk).astype(jnp.int32)
    mmap = jnp.minimum(mb, nblk - 1)
    gmap = jnp.minimum(
        jnp.searchsorted(cum_pc, mmap * BM, side='right').astype(jnp.int32),
        E - 1)
    pos2 = pos.reshape(B, K)
    pos_cat = jnp.concatenate([pos2[:, 0], pos2[:, 1]]).astype(jnp.int32)
    return pos_cat, gmap, mmap, valid


def _sc_dispatch(x, pos_cat):
    tpw = B // NWORK
    ch = 16
    nc = tpw // ch
    mesh = plsc.VectorSubcoreMesh(core_axis_name="c", subcore_axis_name="s")

    @functools.partial(
        pl.kernel,
        out_type=jax.ShapeDtypeStruct((RP, D), jnp.float32),
        mesh=mesh,
        scratch_types=[
            pltpu.VMEM((2, ch, D), jnp.float32),
            pltpu.VMEM((2, ch), jnp.int32),
            pltpu.VMEM((2, ch), jnp.int32),
            pltpu.SemaphoreType.DMA,
            pltpu.SemaphoreType.DMA,
        ],
    )
    def sk(x_hbm, pos_hbm, out_hbm, rows_v, i0_v, i1_v, sem0, sem1):
        wid = lax.axis_index("s") * 2 + lax.axis_index("c")
        base = wid * tpw
        sems = (sem0, sem1)
        pending = [[], []]
        for c in range(nc):
            b = c % 2
            for hnd in pending[b]:
                hnd.wait()
            pending[b] = []
            tb = base + c * ch
            pltpu.sync_copy(x_hbm.at[pl.ds(tb, ch)], rows_v.at[b])
            pltpu.sync_copy(pos_hbm.at[pl.ds(tb, ch)], i0_v.at[b])
            pltpu.sync_copy(pos_hbm.at[pl.ds(B + tb, ch)], i1_v.at[b])
            pending[b].append(pltpu.async_copy(
                rows_v.at[b], out_hbm.at[i0_v.at[b]], sems[b]))
            pending[b].append(pltpu.async_copy(
                rows_v.at[b], out_hbm.at[i1_v.at[b]], sems[b]))
        for plist in pending:
            for hnd in plist:
                hnd.wait()

    return sk(x, pos_cat)


def _sc_gather(table, idx):
    n_idx = idx.shape[0]
    npw = n_idx // NWORK
    ch = npw // NCH
    mesh = plsc.VectorSubcoreMesh(core_axis_name="c", subcore_axis_name="s")

    @functools.partial(
        pl.kernel,
        out_type=jax.ShapeDtypeStruct((n_idx, D), jnp.float32),
        mesh=mesh,
        scratch_types=[
            pltpu.VMEM((npw,), jnp.int32),
            pltpu.VMEM((2, ch, D), jnp.float32),
            pltpu.SemaphoreType.DMA,
            pltpu.SemaphoreType.DMA,
        ],
    )
    def gk(table_hbm, idx_hbm, out_hbm, idx_v, rows_v, sem0, sem1):
        wid = lax.axis_index("s") * 2 + lax.axis_index("c")
        base = wid * npw
        pltpu.sync_copy(idx_hbm.at[pl.ds(base, npw)], idx_v)
        sems = (sem0, sem1)
        pending = pltpu.async_copy(
            table_hbm.at[idx_v.at[pl.ds(0, ch)]], rows_v.at[0], sems[0])
        for ci in range(NCH):
            nxt = None
            if ci + 1 < NCH:
                nxt = pltpu.async_copy(
                    table_hbm.at[idx_v.at[pl.ds((ci + 1) * ch, ch)]],
                    rows_v.at[(ci + 1) % 2], sems[(ci + 1) % 2])
            pending.wait()
            pltpu.sync_copy(rows_v.at[ci % 2],
                            out_hbm.at[pl.ds(base + ci * ch, ch)])
            pending = nxt

    return gk(table, idx)


def _glayer_body(gmap_ref, mmap_ref, valid_ref, x_ref, w_ref, b_ref, out_ref,
                 *, relu):
    mb = pl.program_id(0)
    v = valid_ref[mb] == 1

    @pl.when(v)
    def _():
        y = jnp.dot(x_ref[...], w_ref[0], preferred_element_type=jnp.float32)
        y = y + b_ref[0]
        if relu:
            y = jnp.maximum(y, 0.0)
        out_ref[...] = y


def _glayer(xin, W, b, gmap, mmap, valid, relu):
    n_out = W.shape[2]
    grid_spec = pltpu.PrefetchScalarGridSpec(
        num_scalar_prefetch=3,
        grid=(RB,),
        in_specs=[
            pl.BlockSpec((BM, W.shape[1]), lambda mb, g, m, v: (m[mb], 0)),
            pl.BlockSpec((1, W.shape[1], n_out),
                         lambda mb, g, m, v: (g[mb], 0, 0)),
            pl.BlockSpec((1, 1, n_out), lambda mb, g, m, v: (g[mb], 0, 0)),
        ],
        out_specs=pl.BlockSpec((BM, n_out), lambda mb, g, m, v: (m[mb], 0)),
    )
    return pl.pallas_call(
        functools.partial(_glayer_body, relu=relu),
        grid_spec=grid_spec,
        out_shape=jax.ShapeDtypeStruct((RP, n_out), jnp.float32),
    )(gmap, mmap, valid, xin, W, b.reshape(E, 1, n_out))


def _slayer(xin, W, b, relu, x_3d):
    n_in, n_out = W.shape[1], W.shape[2]
    x_spec = (pl.BlockSpec((1, GB, n_in), lambda s, tb: (s, tb, 0)) if x_3d
              else pl.BlockSpec((GB, n_in), lambda s, tb: (tb, 0)))

    def body(x_ref, w_ref, b_ref, out_ref):
        xb = x_ref[0] if x_3d else x_ref[...]
        y = jnp.dot(xb, w_ref[0], preferred_element_type=jnp.float32)
        y = y + b_ref[0]
        if relu:
            y = jnp.maximum(y, 0.0)
        out_ref[0] = y

    return pl.pallas_call(
        body,
        grid=(ES, TB),
        in_specs=[
            x_spec,
            pl.BlockSpec((1, n_in, n_out), lambda s, tb: (s, 0, 0)),
            pl.BlockSpec((1, 1, n_out), lambda s, tb: (s, 0, 0)),
        ],
        out_specs=pl.BlockSpec((1, GB, n_out), lambda s, tb: (s, tb, 0)),
        out_shape=jax.ShapeDtypeStruct((ES, B, n_out), jnp.float32),
    )(xin, W, b.reshape(ES, 1, n_out))


def _final_body(sc_ref, tw_ref, y0_ref, y1_ref, g0_ref, g1_ref, out_ref):
    sw0 = sc_ref[...][:, 0:1]
    sw1 = sc_ref[...][:, 1:2]
    w0 = tw_ref[...][:, 0:1]
    w1 = tw_ref[...][:, 1:2]
    out_ref[...] = (sw0 * y0_ref[0] + sw1 * y1_ref[0]
                    + w0 * g0_ref[...] + w1 * g1_ref[...])


def _final(scores, tw, ys, g):
    return pl.pallas_call(
        _final_body,
        grid=(TB,),
        in_specs=[
            pl.BlockSpec((GB, 128), lambda tb: (tb, 0)),
            pl.BlockSpec((GB, 128), lambda tb: (tb, 0)),
            pl.BlockSpec((1, GB, O), lambda tb: (0, tb, 0)),
            pl.BlockSpec((1, GB, O), lambda tb: (1, tb, 0)),
            pl.BlockSpec((GB, O), lambda tb: (tb, 0)),
            pl.BlockSpec((GB, O), lambda tb: (tb + TB, 0)),
        ],
        out_specs=pl.BlockSpec((GB, O), lambda tb: (tb, 0)),
        out_shape=jax.ShapeDtypeStruct((B, O), jnp.float32),
    )(scores, tw, ys, ys, g, g)


def kernel(x, gate_W, gate_b, W1, b1, W2, b2, Ws1, bs1, Ws2, bs2):
    scores, tw, tid = _gate(x, gate_W, gate_b)
    pos_cat, gmap, mmap, valid = _route_meta(tid[:, :K])
    xr = _sc_dispatch(x, pos_cat)
    h1r = _glayer(xr, W1, b1, gmap, mmap, valid, relu=True)
    y2r = _glayer(h1r, W2, b2, gmap, mmap, valid, relu=False)
    g = _sc_gather(y2r, pos_cat)
    hs = _slayer(x, Ws1, bs1, relu=True, x_3d=False)
    ys = _slayer(hs, Ws2, bs2, relu=False, x_3d=True)
    return _final(scores, tw, ys, g)


# speedup vs baseline: 1.0467x; 1.0017x over previous
import functools

import jax
import jax.numpy as jnp
from jax import lax
from jax.experimental import pallas as pl
from jax.experimental.pallas import tpu as pltpu
from jax.experimental.pallas import tpu_sc as plsc

B = 2048
D = 2048
H = 2048
O = 2048
E = 8
ES = 2
K = 2

GB = 256
BM = 256
RP = K * B + E * BM
RB = RP // BM
TB = B // GB

NWORK = 32
NCH = 4


def _gate_body(x_ref, gw_ref, gb_ref, sc_ref, tw_ref, tid_ref):
    xb = x_ref[...]
    logits = jnp.dot(xb, gw_ref[...], preferred_element_type=jnp.float32)
    logits = logits + gb_ref[...]
    c = lax.broadcasted_iota(jnp.int32, (GB, 128), 1)
    lm = jnp.where(c < E + ES, logits, -1e30)
    m = jnp.max(lm, axis=1, keepdims=True)
    p = jnp.exp(lm - m)
    scores = p / jnp.sum(p, axis=1, keepdims=True)
    sc_ref[...] = scores

    es = jnp.where((c >= ES) & (c < E + ES), scores, -1.0)
    m1 = jnp.max(es, axis=1, keepdims=True)
    i1 = jnp.min(jnp.where(es == m1, c, 999), axis=1, keepdims=True)
    es2 = jnp.where(c == i1, -1.0, es)
    m2 = jnp.max(es2, axis=1, keepdims=True)
    i2 = jnp.min(jnp.where(es2 == m2, c, 999), axis=1, keepdims=True)
    tw_ref[...] = jnp.where(c == 0, m1, 0.0) + jnp.where(c == 1, m2, 0.0)
    tid_ref[...] = (jnp.where(c == 0, i1 - ES, 0)
                    + jnp.where(c == 1, i2 - ES, 0))


def _gate(x, gate_W, gate_b):
    gw = jnp.zeros((D, 128), jnp.float32).at[:, :E + ES].set(gate_W)
    gb = jnp.zeros((1, 128), jnp.float32).at[0, :E + ES].set(gate_b)
    return pl.pallas_call(
        _gate_body,
        grid=(TB,),
        in_specs=[
            pl.BlockSpec((GB, D), lambda i: (i, 0)),
            pl.BlockSpec((D, 128), lambda i: (0, 0)),
            pl.BlockSpec((1, 128), lambda i: (0, 0)),
        ],
        out_specs=[
            pl.BlockSpec((GB, 128), lambda i: (i, 0)),
            pl.BlockSpec((GB, 128), lambda i: (i, 0)),
            pl.BlockSpec((GB, 128), lambda i: (i, 0)),
        ],
        out_shape=[
            jax.ShapeDtypeStruct((B, 128), jnp.float32),
            jax.ShapeDtypeStruct((B, 128), jnp.float32),
            jax.ShapeDtypeStruct((B, 128), jnp.int32),
        ],
    )(x, gw, gb)


def _route_meta(tid2):
    e_flat = tid2.reshape(-1)
    oh = (e_flat[:, None] == jnp.arange(E)[None, :]).astype(jnp.int32)
    counts = jnp.sum(oh, axis=0)
    rank_all = jnp.cumsum(oh, axis=0) - oh
    rank = jnp.take_along_axis(rank_all, e_flat[:, None], axis=1)[:, 0]
    pc = ((counts + BM - 1) // BM) * BM
    cum_pc = jnp.cumsum(pc)
    off = cum_pc - pc
    pos = off[e_flat] + rank
    nblk = cum_pc[-1] // BM
    mb = jnp.arange(RB, dtype=jnp.int32)
    valid = (mb < nblk).astype(jnp.int32)
    mmap = jnp.minimum(mb, nblk - 1)
    gmap = jnp.minimum(
        jnp.searchsorted(cum_pc, mmap * BM, side='right').astype(jnp.int32),
        E - 1)
    pos2 = pos.reshape(B, K)
    pos_cat = jnp.concatenate([pos2[:, 0], pos2[:, 1]]).astype(jnp.int32)
    return pos_cat, gmap, mmap, valid


def _sc_dispatch(x, pos_cat):
    tpw = B // NWORK
    ch = 16
    nc = tpw // ch
    mesh = plsc.VectorSubcoreMesh(core_axis_name="c", subcore_axis_name="s")

    @functools.partial(
        pl.kernel,
        out_type=jax.ShapeDtypeStruct((RP, D), jnp.float32),
        mesh=mesh,
        scratch_types=[
            pltpu.VMEM((2, ch, D), jnp.float32),
            pltpu.VMEM((2, ch), jnp.int32),
            pltpu.VMEM((2, ch), jnp.int32),
            pltpu.SemaphoreType.DMA,
            pltpu.SemaphoreType.DMA,
        ],
    )
    def sk(x_hbm, pos_hbm, out_hbm, rows_v, i0_v, i1_v, sem0, sem1):
        wid = lax.axis_index("s") * 2 + lax.axis_index("c")
        base = wid * tpw
        sems = (sem0, sem1)
        pending = [[], []]
        for c in range(nc):
            b = c % 2
            for hnd in pending[b]:
                hnd.wait()
            pending[b] = []
            tb = base + c * ch
            pltpu.sync_copy(x_hbm.at[pl.ds(tb, ch)], rows_v.at[b])
            pltpu.sync_copy(pos_hbm.at[pl.ds(tb, ch)], i0_v.at[b])
            pltpu.sync_copy(pos_hbm.at[pl.ds(B + tb, ch)], i1_v.at[b])
            pending[b].append(pltpu.async_copy(
                rows_v.at[b], out_hbm.at[i0_v.at[b]], sems[b]))
            pending[b].append(pltpu.async_copy(
                rows_v.at[b], out_hbm.at[i1_v.at[b]], sems[b]))
        for plist in pending:
            for hnd in plist:
                hnd.wait()

    return sk(x, pos_cat)


def _sc_gather(table, idx):
    n_idx = idx.shape[0]
    npw = n_idx // NWORK
    ch = npw // NCH
    mesh = plsc.VectorSubcoreMesh(core_axis_name="c", subcore_axis_name="s")

    @functools.partial(
        pl.kernel,
        out_type=jax.ShapeDtypeStruct((n_idx, D), jnp.float32),
        mesh=mesh,
        scratch_types=[
            pltpu.VMEM((npw,), jnp.int32),
            pltpu.VMEM((1, ch, D), jnp.float32),
            pltpu.SemaphoreType.DMA,
        ],
    )
    def gk(table_hbm, idx_hbm, out_hbm, idx_v, rows_v, sem0):
        wid = lax.axis_index("s") * 2 + lax.axis_index("c")
        base = wid * npw
        pltpu.sync_copy(idx_hbm.at[pl.ds(base, npw)], idx_v)
        for ci in range(NCH):
            pltpu.async_copy(
                table_hbm.at[idx_v.at[pl.ds(ci * ch, ch)]],
                rows_v.at[0], sem0).wait()
            pltpu.sync_copy(rows_v.at[0],
                            out_hbm.at[pl.ds(base + ci * ch, ch)])

    return gk(table, idx)


def _glayer_body(gmap_ref, mmap_ref, valid_ref, x_ref, w_ref, b_ref, out_ref,
                 *, relu):
    mb = pl.program_id(0)
    v = valid_ref[mb] == 1

    @pl.when(v)
    def _():
        if relu:
            y = jnp.dot(x_ref[...].astype(jnp.bfloat16),
                        w_ref[0].astype(jnp.bfloat16),
                        preferred_element_type=jnp.float32)
        else:
            y = jnp.dot(x_ref[...], w_ref[0],
                        preferred_element_type=jnp.float32)
        y = y + b_ref[0]
        if relu:
            y = jnp.maximum(y, 0.0)
        out_ref[...] = y


def _glayer(xin, W, b, gmap, mmap, valid, relu):
    n_out = W.shape[2]
    grid_spec = pltpu.PrefetchScalarGridSpec(
        num_scalar_prefetch=3,
        grid=(RB,),
        in_specs=[
            pl.BlockSpec((BM, W.shape[1]), lambda mb, g, m, v: (m[mb], 0)),
            pl.BlockSpec((1, W.shape[1], n_out),
                         lambda mb, g, m, v: (g[mb], 0, 0)),
            pl.BlockSpec((1, 1, n_out), lambda mb, g, m, v: (g[mb], 0, 0)),
        ],
        out_specs=pl.BlockSpec((BM, n_out), lambda mb, g, m, v: (m[mb], 0)),
    )
    return pl.pallas_call(
        functools.partial(_glayer_body, relu=relu),
        grid_spec=grid_spec,
        out_shape=jax.ShapeDtypeStruct((RP, n_out), jnp.float32),
    )(gmap, mmap, valid, xin, W, b.reshape(E, 1, n_out))


def _slayer(xin, W, b, relu, x_3d):
    n_in, n_out = W.shape[1], W.shape[2]
    x_spec = (pl.BlockSpec((1, GB, n_in), lambda s, tb: (s, tb, 0)) if x_3d
              else pl.BlockSpec((GB, n_in), lambda s, tb: (tb, 0)))

    def body(x_ref, w_ref, b_ref, out_ref):
        xb = x_ref[0] if x_3d else x_ref[...]
        if relu:
            y = jnp.dot(xb.astype(jnp.bfloat16), w_ref[0].astype(jnp.bfloat16),
                        preferred_element_type=jnp.float32)
        else:
            y = jnp.dot(xb, w_ref[0], preferred_element_type=jnp.float32)
        y = y + b_ref[0]
        if relu:
            y = jnp.maximum(y, 0.0)
        out_ref[0] = y

    return pl.pallas_call(
        body,
        grid=(ES, TB),
        in_specs=[
            x_spec,
            pl.BlockSpec((1, n_in, n_out), lambda s, tb: (s, 0, 0)),
            pl.BlockSpec((1, 1, n_out), lambda s, tb: (s, 0, 0)),
        ],
        out_specs=pl.BlockSpec((1, GB, n_out), lambda s, tb: (s, tb, 0)),
        out_shape=jax.ShapeDtypeStruct((ES, B, n_out), jnp.float32),
    )(xin, W, b.reshape(ES, 1, n_out))


def _final_body(sc_ref, tw_ref, y0_ref, y1_ref, g0_ref, g1_ref, out_ref):
    sw0 = sc_ref[...][:, 0:1]
    sw1 = sc_ref[...][:, 1:2]
    w0 = tw_ref[...][:, 0:1]
    w1 = tw_ref[...][:, 1:2]
    out_ref[...] = (sw0 * y0_ref[0] + sw1 * y1_ref[0]
                    + w0 * g0_ref[...] + w1 * g1_ref[...])


def _final(scores, tw, ys, g):
    return pl.pallas_call(
        _final_body,
        grid=(TB,),
        in_specs=[
            pl.BlockSpec((GB, 128), lambda tb: (tb, 0)),
            pl.BlockSpec((GB, 128), lambda tb: (tb, 0)),
            pl.BlockSpec((1, GB, O), lambda tb: (0, tb, 0)),
            pl.BlockSpec((1, GB, O), lambda tb: (1, tb, 0)),
            pl.BlockSpec((GB, O), lambda tb: (tb, 0)),
            pl.BlockSpec((GB, O), lambda tb: (tb + TB, 0)),
        ],
        out_specs=pl.BlockSpec((GB, O), lambda tb: (tb, 0)),
        out_shape=jax.ShapeDtypeStruct((B, O), jnp.float32),
    )(scores, tw, ys, ys, g, g)


def kernel(x, gate_W, gate_b, W1, b1, W2, b2, Ws1, bs1, Ws2, bs2):
    scores, tw, tid = _gate(x, gate_W, gate_b)
    pos_cat, gmap, mmap, valid = _route_meta(tid[:, :K])
    xr = _sc_dispatch(x, pos_cat)
    h1r = _glayer(xr, W1, b1, gmap, mmap, valid, relu=True)
    y2r = _glayer(h1r, W2, b2, gmap, mmap, valid, relu=False)
    g = _sc_gather(y2r, pos_cat)
    hs = _slayer(x, Ws1, bs1, relu=True, x_3d=False)
    ys = _slayer(hs, Ws2, bs2, relu=False, x_3d=True)
    return _final(scores, tw, ys, g)


# speedup vs baseline: 1.0489x; 1.0020x over previous
import functools

import jax
import jax.numpy as jnp
from jax import lax
from jax.experimental import pallas as pl
from jax.experimental.pallas import tpu as pltpu
from jax.experimental.pallas import tpu_sc as plsc

B = 2048
D = 2048
H = 2048
O = 2048
E = 8
ES = 2
K = 2

GB = 256
BM = 256
RP = K * B + E * BM
RB = RP // BM
TB = B // GB

NWORK = 32
NCH = 4


def _gate_body(x_ref, gw_ref, gb_ref, sc_ref, tw_ref, tid_ref):
    xb = x_ref[...]
    logits = jnp.dot(xb, gw_ref[...], preferred_element_type=jnp.float32)
    logits = logits + gb_ref[...]
    c = lax.broadcasted_iota(jnp.int32, (GB, 128), 1)
    lm = jnp.where(c < E + ES, logits, -1e30)
    m = jnp.max(lm, axis=1, keepdims=True)
    p = jnp.exp(lm - m)
    scores = p / jnp.sum(p, axis=1, keepdims=True)
    sc_ref[...] = scores

    es = jnp.where((c >= ES) & (c < E + ES), scores, -1.0)
    m1 = jnp.max(es, axis=1, keepdims=True)
    i1 = jnp.min(jnp.where(es == m1, c, 999), axis=1, keepdims=True)
    es2 = jnp.where(c == i1, -1.0, es)
    m2 = jnp.max(es2, axis=1, keepdims=True)
    i2 = jnp.min(jnp.where(es2 == m2, c, 999), axis=1, keepdims=True)
    tw_ref[...] = jnp.where(c == 0, m1, 0.0) + jnp.where(c == 1, m2, 0.0)
    tid_ref[...] = (jnp.where(c == 0, i1 - ES, 0)
                    + jnp.where(c == 1, i2 - ES, 0))


def _gate(x, gate_W, gate_b):
    gw = jnp.zeros((D, 128), jnp.float32).at[:, :E + ES].set(gate_W)
    gb = jnp.zeros((1, 128), jnp.float32).at[0, :E + ES].set(gate_b)
    return pl.pallas_call(
        _gate_body,
        grid=(TB,),
        in_specs=[
            pl.BlockSpec((GB, D), lambda i: (i, 0)),
            pl.BlockSpec((D, 128), lambda i: (0, 0)),
            pl.BlockSpec((1, 128), lambda i: (0, 0)),
        ],
        out_specs=[
            pl.BlockSpec((GB, 128), lambda i: (i, 0)),
            pl.BlockSpec((GB, 128), lambda i: (i, 0)),
            pl.BlockSpec((GB, 128), lambda i: (i, 0)),
        ],
        out_shape=[
            jax.ShapeDtypeStruct((B, 128), jnp.float32),
            jax.ShapeDtypeStruct((B, 128), jnp.float32),
            jax.ShapeDtypeStruct((B, 128), jnp.int32),
        ],
    )(x, gw, gb)


def _route_meta(tid2):
    e_flat = tid2.reshape(-1)
    oh = (e_flat[:, None] == jnp.arange(E)[None, :]).astype(jnp.int32)
    counts = jnp.sum(oh, axis=0)
    rank_all = jnp.cumsum(oh, axis=0) - oh
    rank = jnp.take_along_axis(rank_all, e_flat[:, None], axis=1)[:, 0]
    pc = ((counts + BM - 1) // BM) * BM
    cum_pc = jnp.cumsum(pc)
    off = cum_pc - pc
    pos = off[e_flat] + rank
    nblk = cum_pc[-1] // BM
    mb = jnp.arange(RB, dtype=jnp.int32)
    valid = (mb < nblk).astype(jnp.int32)
    mmap = jnp.minimum(mb, nblk - 1)
    gmap = jnp.minimum(
        jnp.searchsorted(cum_pc, mmap * BM, side='right').astype(jnp.int32),
        E - 1)
    pos2 = pos.reshape(B, K)
    pos_cat = jnp.concatenate([pos2[:, 0], pos2[:, 1]]).astype(jnp.int32)
    return pos_cat, gmap, mmap, valid


def _sc_dispatch(x, pos_cat):
    tpw = B // NWORK
    ch = 16
    nc = tpw // ch
    mesh = plsc.VectorSubcoreMesh(core_axis_name="c", subcore_axis_name="s")

    @functools.partial(
        pl.kernel,
        out_type=jax.ShapeDtypeStruct((RP, D), jnp.float32),
        mesh=mesh,
        scratch_types=[
            pltpu.VMEM((2, ch, D), jnp.float32),
            pltpu.VMEM((2, ch), jnp.int32),
            pltpu.VMEM((2, ch), jnp.int32),
            pltpu.SemaphoreType.DMA,
            pltpu.SemaphoreType.DMA,
        ],
    )
    def sk(x_hbm, pos_hbm, out_hbm, rows_v, i0_v, i1_v, sem0, sem1):
        wid = lax.axis_index("s") * 2 + lax.axis_index("c")
        base = wid * tpw
        sems = (sem0, sem1)
        pending = [[], []]
        for c in range(nc):
            b = c % 2
            for hnd in pending[b]:
                hnd.wait()
            pending[b] = []
            tb = base + c * ch
            pltpu.sync_copy(x_hbm.at[pl.ds(tb, ch)], rows_v.at[b])
            pltpu.sync_copy(pos_hbm.at[pl.ds(tb, ch)], i0_v.at[b])
            pltpu.sync_copy(pos_hbm.at[pl.ds(B + tb, ch)], i1_v.at[b])
            pending[b].append(pltpu.async_copy(
                rows_v.at[b], out_hbm.at[i0_v.at[b]], sems[b]))
            pending[b].append(pltpu.async_copy(
                rows_v.at[b], out_hbm.at[i1_v.at[b]], sems[b]))
        for plist in pending:
            for hnd in plist:
                hnd.wait()

    return sk(x, pos_cat)


def _sc_gather(table, idx):
    n_idx = idx.shape[0]
    npw = n_idx // NWORK
    ch = npw // NCH
    mesh = plsc.VectorSubcoreMesh(core_axis_name="c", subcore_axis_name="s")

    @functools.partial(
        pl.kernel,
        out_type=jax.ShapeDtypeStruct((n_idx, D), jnp.float32),
        mesh=mesh,
        scratch_types=[
            pltpu.VMEM((npw,), jnp.int32),
            pltpu.VMEM((1, ch, D), jnp.float32),
            pltpu.SemaphoreType.DMA,
        ],
    )
    def gk(table_hbm, idx_hbm, out_hbm, idx_v, rows_v, sem0):
        wid = lax.axis_index("s") * 2 + lax.axis_index("c")
        base = wid * npw
        pltpu.sync_copy(idx_hbm.at[pl.ds(base, npw)], idx_v)
        for ci in range(NCH):
            pltpu.async_copy(
                table_hbm.at[idx_v.at[pl.ds(ci * ch, ch)]],
                rows_v.at[0], sem0).wait()
            pltpu.sync_copy(rows_v.at[0],
                            out_hbm.at[pl.ds(base + ci * ch, ch)])

    return gk(table, idx)


def _glayer_body(gmap_ref, mmap_ref, valid_ref, x_ref, w_ref, b_ref, out_ref,
                 *, relu):
    mb = pl.program_id(0)
    v = valid_ref[mb] == 1

    @pl.when(v)
    def _():
        y = jnp.dot(x_ref[...], w_ref[0], preferred_element_type=jnp.float32)
        y = y + b_ref[0]
        if relu:
            y = jnp.maximum(y, 0.0)
        out_ref[...] = y


def _glayer(xin, W, b, gmap, mmap, valid, relu):
    n_out = W.shape[2]
    grid_spec = pltpu.PrefetchScalarGridSpec(
        num_scalar_prefetch=3,
        grid=(RB,),
        in_specs=[
            pl.BlockSpec((BM, W.shape[1]), lambda mb, g, m, v: (m[mb], 0)),
            pl.BlockSpec((1, W.shape[1], n_out),
                         lambda mb, g, m, v: (g[mb], 0, 0)),
            pl.BlockSpec((1, 1, n_out), lambda mb, g, m, v: (g[mb], 0, 0)),
        ],
        out_specs=pl.BlockSpec((BM, n_out), lambda mb, g, m, v: (m[mb], 0)),
    )
    return pl.pallas_call(
        functools.partial(_glayer_body, relu=relu),
        grid_spec=grid_spec,
        out_shape=jax.ShapeDtypeStruct((RP, n_out), jnp.float32),
    )(gmap, mmap, valid, xin, W, b.reshape(E, 1, n_out))


def _slayer(xin, W, b, relu, x_3d):
    n_in, n_out = W.shape[1], W.shape[2]
    x_spec = (pl.BlockSpec((1, GB, n_in), lambda s, tb: (s, tb, 0)) if x_3d
              else pl.BlockSpec((GB, n_in), lambda s, tb: (tb, 0)))

    def body(x_ref, w_ref, b_ref, out_ref):
        xb = x_ref[0] if x_3d else x_ref[...]
        y = jnp.dot(xb, w_ref[0], preferred_element_type=jnp.float32)
        y = y + b_ref[0]
        if relu:
            y = jnp.maximum(y, 0.0)
        out_ref[0] = y

    return pl.pallas_call(
        body,
        grid=(ES, TB),
        in_specs=[
            x_spec,
            pl.BlockSpec((1, n_in, n_out), lambda s, tb: (s, 0, 0)),
            pl.BlockSpec((1, 1, n_out), lambda s, tb: (s, 0, 0)),
        ],
        out_specs=pl.BlockSpec((1, GB, n_out), lambda s, tb: (s, tb, 0)),
        out_shape=jax.ShapeDtypeStruct((ES, B, n_out), jnp.float32),
    )(xin, W, b.reshape(ES, 1, n_out))


def _final_body(sc_ref, tw_ref, y0_ref, y1_ref, g0_ref, g1_ref, out_ref):
    sw0 = sc_ref[...][:, 0:1]
    sw1 = sc_ref[...][:, 1:2]
    w0 = tw_ref[...][:, 0:1]
    w1 = tw_ref[...][:, 1:2]
    out_ref[...] = (sw0 * y0_ref[0] + sw1 * y1_ref[0]
                    + w0 * g0_ref[...] + w1 * g1_ref[...])


def _final(scores, tw, ys, g):
    return pl.pallas_call(
        _final_body,
        grid=(TB,),
        in_specs=[
            pl.BlockSpec((GB, 128), lambda tb: (tb, 0)),
            pl.BlockSpec((GB, 128), lambda tb: (tb, 0)),
            pl.BlockSpec((1, GB, O), lambda tb: (0, tb, 0)),
            pl.BlockSpec((1, GB, O), lambda tb: (1, tb, 0)),
            pl.BlockSpec((GB, O), lambda tb: (tb, 0)),
            pl.BlockSpec((GB, O), lambda tb: (tb + TB, 0)),
        ],
        out_specs=pl.BlockSpec((GB, O), lambda tb: (tb, 0)),
        out_shape=jax.ShapeDtypeStruct((B, O), jnp.float32),
    )(scores, tw, ys, ys, g, g)


def kernel(x, gate_W, gate_b, W1, b1, W2, b2, Ws1, bs1, Ws2, bs2):
    scores, tw, tid = _gate(x, gate_W, gate_b)
    pos_cat, gmap, mmap, valid = _route_meta(tid[:, :K])
    xr = _sc_dispatch(x, pos_cat)
    h1r = _glayer(xr, W1, b1, gmap, mmap, valid, relu=True)
    y2r = _glayer(h1r, W2, b2, gmap, mmap, valid, relu=False)
    g = _sc_gather(y2r, pos_cat)
    hs = _slayer(x, Ws1, bs1, relu=True, x_3d=False)
    ys = _slayer(hs, Ws2, bs2, relu=False, x_3d=True)
    return _final(scores, tw, ys, g)


# speedup vs baseline: 1.0590x; 1.0097x over previous
import functools

import jax
import jax.numpy as jnp
from jax import lax
from jax.experimental import pallas as pl
from jax.experimental.pallas import tpu as pltpu
from jax.experimental.pallas import tpu_sc as plsc

B = 2048
D = 2048
H = 2048
O = 2048
E = 8
ES = 2
K = 2

GB = 512
BM = 256
RP = K * B + E * BM
RB = RP // BM
TB = B // GB

NWORK = 32
NCH = 4


def _gate_body(x_ref, gw_ref, gb_ref, sc_ref, tw_ref, tid_ref):
    xb = x_ref[...]
    logits = jnp.dot(xb, gw_ref[...], preferred_element_type=jnp.float32)
    logits = logits + gb_ref[...]
    c = lax.broadcasted_iota(jnp.int32, (GB, 128), 1)
    lm = jnp.where(c < E + ES, logits, -1e30)
    m = jnp.max(lm, axis=1, keepdims=True)
    p = jnp.exp(lm - m)
    scores = p / jnp.sum(p, axis=1, keepdims=True)
    sc_ref[...] = scores

    es = jnp.where((c >= ES) & (c < E + ES), scores, -1.0)
    m1 = jnp.max(es, axis=1, keepdims=True)
    i1 = jnp.min(jnp.where(es == m1, c, 999), axis=1, keepdims=True)
    es2 = jnp.where(c == i1, -1.0, es)
    m2 = jnp.max(es2, axis=1, keepdims=True)
    i2 = jnp.min(jnp.where(es2 == m2, c, 999), axis=1, keepdims=True)
    tw_ref[...] = jnp.where(c == 0, m1, 0.0) + jnp.where(c == 1, m2, 0.0)
    tid_ref[...] = (jnp.where(c == 0, i1 - ES, 0)
                    + jnp.where(c == 1, i2 - ES, 0))


def _gate(x, gate_W, gate_b):
    gw = jnp.zeros((D, 128), jnp.float32).at[:, :E + ES].set(gate_W)
    gb = jnp.zeros((1, 128), jnp.float32).at[0, :E + ES].set(gate_b)
    return pl.pallas_call(
        _gate_body,
        grid=(TB,),
        in_specs=[
            pl.BlockSpec((GB, D), lambda i: (i, 0)),
            pl.BlockSpec((D, 128), lambda i: (0, 0)),
            pl.BlockSpec((1, 128), lambda i: (0, 0)),
        ],
        out_specs=[
            pl.BlockSpec((GB, 128), lambda i: (i, 0)),
            pl.BlockSpec((GB, 128), lambda i: (i, 0)),
            pl.BlockSpec((GB, 128), lambda i: (i, 0)),
        ],
        out_shape=[
            jax.ShapeDtypeStruct((B, 128), jnp.float32),
            jax.ShapeDtypeStruct((B, 128), jnp.float32),
            jax.ShapeDtypeStruct((B, 128), jnp.int32),
        ],
    )(x, gw, gb)


def _route_meta(tid2):
    e_flat = tid2.reshape(-1)
    oh = (e_flat[:, None] == jnp.arange(E)[None, :]).astype(jnp.int32)
    counts = jnp.sum(oh, axis=0)
    rank_all = jnp.cumsum(oh, axis=0) - oh
    rank = jnp.take_along_axis(rank_all, e_flat[:, None], axis=1)[:, 0]
    pc = ((counts + BM - 1) // BM) * BM
    cum_pc = jnp.cumsum(pc)
    off = cum_pc - pc
    pos = off[e_flat] + rank
    nblk = cum_pc[-1] // BM
    mb = jnp.arange(RB, dtype=jnp.int32)
    valid = (mb < nblk).astype(jnp.int32)
    mmap = jnp.minimum(mb, nblk - 1)
    gmap = jnp.minimum(
        jnp.searchsorted(cum_pc, mmap * BM, side='right').astype(jnp.int32),
        E - 1)
    pos2 = pos.reshape(B, K)
    pos_cat = jnp.concatenate([pos2[:, 0], pos2[:, 1]]).astype(jnp.int32)
    return pos_cat, gmap, mmap, valid


def _sc_dispatch(x, pos_cat):
    tpw = B // NWORK
    ch = 16
    nc = tpw // ch
    mesh = plsc.VectorSubcoreMesh(core_axis_name="c", subcore_axis_name="s")

    @functools.partial(
        pl.kernel,
        out_type=jax.ShapeDtypeStruct((RP, D), jnp.float32),
        mesh=mesh,
        scratch_types=[
            pltpu.VMEM((2, ch, D), jnp.float32),
            pltpu.VMEM((2, ch), jnp.int32),
            pltpu.VMEM((2, ch), jnp.int32),
            pltpu.SemaphoreType.DMA,
            pltpu.SemaphoreType.DMA,
        ],
    )
    def sk(x_hbm, pos_hbm, out_hbm, rows_v, i0_v, i1_v, sem0, sem1):
        wid = lax.axis_index("s") * 2 + lax.axis_index("c")
        base = wid * tpw
        sems = (sem0, sem1)
        pending = [[], []]
        for c in range(nc):
            b = c % 2
            for hnd in pending[b]:
                hnd.wait()
            pending[b] = []
            tb = base + c * ch
            pltpu.sync_copy(x_hbm.at[pl.ds(tb, ch)], rows_v.at[b])
            pltpu.sync_copy(pos_hbm.at[pl.ds(tb, ch)], i0_v.at[b])
            pltpu.sync_copy(pos_hbm.at[pl.ds(B + tb, ch)], i1_v.at[b])
            pending[b].append(pltpu.async_copy(
                rows_v.at[b], out_hbm.at[i0_v.at[b]], sems[b]))
            pending[b].append(pltpu.async_copy(
                rows_v.at[b], out_hbm.at[i1_v.at[b]], sems[b]))
        for plist in pending:
            for hnd in plist:
                hnd.wait()

    return sk(x, pos_cat)


def _sc_gather(table, idx):
    n_idx = idx.shape[0]
    npw = n_idx // NWORK
    ch = npw // NCH
    mesh = plsc.VectorSubcoreMesh(core_axis_name="c", subcore_axis_name="s")

    @functools.partial(
        pl.kernel,
        out_type=jax.ShapeDtypeStruct((n_idx, D), jnp.float32),
        mesh=mesh,
        scratch_types=[
            pltpu.VMEM((npw,), jnp.int32),
            pltpu.VMEM((1, ch, D), jnp.float32),
            pltpu.SemaphoreType.DMA,
        ],
    )
    def gk(table_hbm, idx_hbm, out_hbm, idx_v, rows_v, sem0):
        wid = lax.axis_index("s") * 2 + lax.axis_index("c")
        base = wid * npw
        pltpu.sync_copy(idx_hbm.at[pl.ds(base, npw)], idx_v)
        for ci in range(NCH):
            pltpu.async_copy(
                table_hbm.at[idx_v.at[pl.ds(ci * ch, ch)]],
                rows_v.at[0], sem0).wait()
            pltpu.sync_copy(rows_v.at[0],
                            out_hbm.at[pl.ds(base + ci * ch, ch)])

    return gk(table, idx)


def _glayer_body(gmap_ref, mmap_ref, valid_ref, x_ref, w_ref, b_ref, out_ref,
                 *, relu):
    mb = pl.program_id(0)
    v = valid_ref[mb] == 1

    @pl.when(v)
    def _():
        y = jnp.dot(x_ref[...], w_ref[0], preferred_element_type=jnp.float32)
        y = y + b_ref[0]
        if relu:
            y = jnp.maximum(y, 0.0)
        out_ref[...] = y


def _glayer(xin, W, b, gmap, mmap, valid, relu):
    n_out = W.shape[2]
    grid_spec = pltpu.PrefetchScalarGridSpec(
        num_scalar_prefetch=3,
        grid=(RB,),
        in_specs=[
            pl.BlockSpec((BM, W.shape[1]), lambda mb, g, m, v: (m[mb], 0)),
            pl.BlockSpec((1, W.shape[1], n_out),
                         lambda mb, g, m, v: (g[mb], 0, 0)),
            pl.BlockSpec((1, 1, n_out), lambda mb, g, m, v: (g[mb], 0, 0)),
        ],
        out_specs=pl.BlockSpec((BM, n_out), lambda mb, g, m, v: (m[mb], 0)),
    )
    return pl.pallas_call(
        functools.partial(_glayer_body, relu=relu),
        grid_spec=grid_spec,
        out_shape=jax.ShapeDtypeStruct((RP, n_out), jnp.float32),
    )(gmap, mmap, valid, xin, W, b.reshape(E, 1, n_out))


def _slayer(xin, W, b, relu, x_3d):
    n_in, n_out = W.shape[1], W.shape[2]
    x_spec = (pl.BlockSpec((1, GB, n_in), lambda s, tb: (s, tb, 0)) if x_3d
              else pl.BlockSpec((GB, n_in), lambda s, tb: (tb, 0)))

    def body(x_ref, w_ref, b_ref, out_ref):
        xb = x_ref[0] if x_3d else x_ref[...]
        y = jnp.dot(xb, w_ref[0], preferred_element_type=jnp.float32)
        y = y + b_ref[0]
        if relu:
            y = jnp.maximum(y, 0.0)
        out_ref[0] = y

    return pl.pallas_call(
        body,
        grid=(ES, TB),
        in_specs=[
            x_spec,
            pl.BlockSpec((1, n_in, n_out), lambda s, tb: (s, 0, 0)),
            pl.BlockSpec((1, 1, n_out), lambda s, tb: (s, 0, 0)),
        ],
        out_specs=pl.BlockSpec((1, GB, n_out), lambda s, tb: (s, tb, 0)),
        out_shape=jax.ShapeDtypeStruct((ES, B, n_out), jnp.float32),
    )(xin, W, b.reshape(ES, 1, n_out))


def _final_body(sc_ref, tw_ref, y0_ref, y1_ref, g0_ref, g1_ref, out_ref):
    sw0 = sc_ref[...][:, 0:1]
    sw1 = sc_ref[...][:, 1:2]
    w0 = tw_ref[...][:, 0:1]
    w1 = tw_ref[...][:, 1:2]
    out_ref[...] = (sw0 * y0_ref[0] + sw1 * y1_ref[0]
                    + w0 * g0_ref[...] + w1 * g1_ref[...])


def _final(scores, tw, ys, g):
    return pl.pallas_call(
        _final_body,
        grid=(TB,),
        in_specs=[
            pl.BlockSpec((GB, 128), lambda tb: (tb, 0)),
            pl.BlockSpec((GB, 128), lambda tb: (tb, 0)),
            pl.BlockSpec((1, GB, O), lambda tb: (0, tb, 0)),
            pl.BlockSpec((1, GB, O), lambda tb: (1, tb, 0)),
            pl.BlockSpec((GB, O), lambda tb: (tb, 0)),
            pl.BlockSpec((GB, O), lambda tb: (tb + TB, 0)),
        ],
        out_specs=pl.BlockSpec((GB, O), lambda tb: (tb, 0)),
        out_shape=jax.ShapeDtypeStruct((B, O), jnp.float32),
    )(scores, tw, ys, ys, g, g)


def kernel(x, gate_W, gate_b, W1, b1, W2, b2, Ws1, bs1, Ws2, bs2):
    scores, tw, tid = _gate(x, gate_W, gate_b)
    pos_cat, gmap, mmap, valid = _route_meta(tid[:, :K])
    xr = _sc_dispatch(x, pos_cat)
    h1r = _glayer(xr, W1, b1, gmap, mmap, valid, relu=True)
    y2r = _glayer(h1r, W2, b2, gmap, mmap, valid, relu=False)
    g = _sc_gather(y2r, pos_cat)
    hs = _slayer(x, Ws1, bs1, relu=True, x_3d=False)
    ys = _slayer(hs, Ws2, bs2, relu=False, x_3d=True)
    return _final(scores, tw, ys, g)
